# baseline (speedup 1.0000x reference)
import functools

import jax
import jax.numpy as jnp
from jax import lax
from jax.experimental import pallas as pl
from jax.experimental.pallas import tpu as pltpu
import jax.experimental.pallas.tpu_sc as plsc

N_NODES = 10000
N_PAD = 10240
D_FEAT = 128
HIDDEN = 128
D_HALF = HIDDEN // 2
N_CLASSES = 40
C_PAD = 48

N_SUB = 16
NW = 2 * N_SUB
K_EDGE = 128
G_CH = 8
CHUNKS = 80
E_PAD = NW * K_EDGE * CHUNKS
CHUNKS_SPLIT = E_PAD // (N_SUB * K_EDGE)

ROWS_PER_SUB = N_PAD // N_SUB


def _zero_acc_stripe(rows, acc, sid, d):
    zero16 = jnp.zeros((16,), jnp.float32)

    def zrow(i, carry):
        for j in range(d // 16):
            rows[0, i, pl.ds(j * 16, 16)] = zero16
        return carry

    lax.fori_loop(0, K_EDGE, zrow, 0)
    for t in range(ROWS_PER_SUB // K_EDGE):
        pltpu.sync_copy(
            rows.at[0],
            acc.at[pl.ds(sid * ROWS_PER_SUB + t * K_EDGE, K_EDGE)],
        )


def _edge_loop(chunks, d, eid, h_sp, acc, dst_hbm, ew_hbm,
               isrc, dbuf, wbuf, rows, gsems, msems, wsems, ssems):
    ngroups = chunks // G_CH

    pltpu.sync_copy(dst_hbm.at[eid, 0], dbuf.at[0])
    pltpu.sync_copy(ew_hbm.at[eid, 0], wbuf.at[0])
    pltpu.async_copy(h_sp.at[isrc.at[0]], rows.at[0], gsems.at[0])

    @pl.loop(0, ngroups, step=2)
    def group2(g0):
        for gb in range(2):
            g = g0 + gb

            @pl.when(g > 0)
            def _():
                pltpu.make_async_copy(
                    dst_hbm.at[eid, g], dbuf.at[gb], msems.at[gb]
                ).wait()
                pltpu.make_async_copy(
                    ew_hbm.at[eid, g], wbuf.at[gb], wsems.at[gb]
                ).wait()

            for cc in range(G_CH):
                c = g * G_CH + cc
                b = cc % 2

                pcc, pgb = (cc - 1, gb) if cc > 0 else (G_CH - 1, 1 - gb)

                @pl.when(c > 0)
                def _():
                    pltpu.make_async_copy(
                        rows.at[1 - b], acc.at[dbuf.at[pgb, pcc]], ssems.at[1 - b]
                    ).wait()

                if cc == 0:
                    @pl.when(g + 1 < ngroups)
                    def _():
                        pltpu.async_copy(
                            dst_hbm.at[eid, g + 1], dbuf.at[1 - gb], msems.at[1 - gb]
                        )
                        pltpu.async_copy(
                            ew_hbm.at[eid, g + 1], wbuf.at[1 - gb], wsems.at[1 - gb]
                        )

                @pl.when(c + 1 < chunks)
                def _():
                    pltpu.async_copy(
                        h_sp.at[isrc.at[c + 1]], rows.at[1 - b], gsems.at[1 - b]
                    )

                pltpu.make_async_copy(
                    h_sp.at[isrc.at[c]], rows.at[b], gsems.at[b]
                ).wait()

                def scale16(grp, carry):
                    wv = wbuf[gb, cc, pl.ds(grp * 16, 16)]
                    for ii in range(16):
                        wb = wv.at[jnp.full((16,), ii, jnp.int32)].get(
                            mode="promise_in_bounds"
                        )
                        for j in range(d // 16):
                            rows[b, grp * 16 + ii, pl.ds(j * 16, 16)] = (
                                rows[b, grp * 16 + ii, pl.ds(j * 16, 16)] * wb
                            )
                    return carry

                lax.fori_loop(0, K_EDGE // 16, scale16, 0)

                pltpu.async_copy(
                    rows.at[b], acc.at[dbuf.at[gb, cc]], ssems.at[b], add=True
                )

    lb = (G_CH - 1) % 2
    lgb = (ngroups - 1) % 2
    pltpu.make_async_copy(
        rows.at[lb], acc.at[dbuf.at[lgb, G_CH - 1]], ssems.at[lb]
    ).wait()


def _edge_scratch(d, chunks):
    return [
        pltpu.VMEM((chunks, K_EDGE), jnp.int32),
        pltpu.VMEM((2, G_CH, K_EDGE), jnp.int32),
        pltpu.VMEM((2, G_CH, K_EDGE), jnp.float32),
        pltpu.VMEM((2, K_EDGE, d), jnp.float32),
        pltpu.SemaphoreType.DMA((2,)),
        pltpu.SemaphoreType.DMA((2,)),
        pltpu.SemaphoreType.DMA((2,)),
        pltpu.SemaphoreType.DMA((2,)),
    ]


def _make_edge_agg_split(dch, chunks, tc_tiling):
    mesh = plsc.VectorSubcoreMesh(core_axis_name="c", subcore_axis_name="s")

    @functools.partial(
        pl.kernel,
        out_type=jax.ShapeDtypeStruct((2, N_PAD, dch), jnp.float32),
        mesh=mesh,
        scratch_types=[
            pltpu.VMEM_SHARED((N_PAD, dch), jnp.float32),
            pltpu.VMEM_SHARED((N_PAD, dch), jnp.float32),
        ] + _edge_scratch(dch, chunks),
        compiler_params=pltpu.CompilerParams(
            needs_layout_passes=False, use_tc_tiling_on_sc=tc_tiling
        ),
    )
    def agg(h_hbm, src_hbm, dst_hbm, ew_hbm, out_hbm,
            acc, h_sp, isrc, dbuf, wbuf, rows, gsems, msems, wsems, ssems):
        cid = lax.axis_index("c")
        sid = lax.axis_index("s")

        pltpu.sync_copy(src_hbm.at[sid], isrc)
        pltpu.sync_copy(
            h_hbm.at[cid, pl.ds(sid * ROWS_PER_SUB, ROWS_PER_SUB)],
            h_sp.at[pl.ds(sid * ROWS_PER_SUB, ROWS_PER_SUB)],
        )
        _zero_acc_stripe(rows, acc, sid, dch)
        plsc.subcore_barrier()

        _edge_loop(chunks, dch, sid, h_sp, acc, dst_hbm, ew_hbm,
                   isrc, dbuf, wbuf, rows, gsems, msems, wsems, ssems)

        plsc.subcore_barrier()
        pltpu.sync_copy(
            acc.at[pl.ds(sid * ROWS_PER_SUB, ROWS_PER_SUB)],
            out_hbm.at[cid, pl.ds(sid * ROWS_PER_SUB, ROWS_PER_SUB)],
        )

    return agg


def _make_edge_agg(d, chunks, tc_tiling):
    mesh = plsc.VectorSubcoreMesh(core_axis_name="c", subcore_axis_name="s")

    @functools.partial(
        pl.kernel,
        out_type=jax.ShapeDtypeStruct((2, N_PAD, d), jnp.float32),
        mesh=mesh,
        scratch_types=[
            pltpu.VMEM_SHARED((N_PAD, d), jnp.float32),
            pltpu.VMEM_SHARED((N_PAD, d), jnp.float32),
        ] + _edge_scratch(d, chunks),
        compiler_params=pltpu.CompilerParams(
            needs_layout_passes=False, use_tc_tiling_on_sc=tc_tiling
        ),
    )
    def agg(h_hbm, src_hbm, dst_hbm, ew_hbm, out_hbm,
            acc, h_sp, isrc, dbuf, wbuf, rows, gsems, msems, wsems, ssems):
        cid = lax.axis_index("c")
        sid = lax.axis_index("s")
        wid = cid * N_SUB + sid

        pltpu.sync_copy(src_hbm.at[wid], isrc)
        pltpu.sync_copy(
            h_hbm.at[pl.ds(sid * ROWS_PER_SUB, ROWS_PER_SUB)],
            h_sp.at[pl.ds(sid * ROWS_PER_SUB, ROWS_PER_SUB)],
        )
        _zero_acc_stripe(rows, acc, sid, d)
        plsc.subcore_barrier()

        _edge_loop(chunks, d, wid, h_sp, acc, dst_hbm, ew_hbm,
                   isrc, dbuf, wbuf, rows, gsems, msems, wsems, ssems)

        plsc.subcore_barrier()
        pltpu.sync_copy(
            acc.at[pl.ds(sid * ROWS_PER_SUB, ROWS_PER_SUB)],
            out_hbm.at[cid, pl.ds(sid * ROWS_PER_SUB, ROWS_PER_SUB)],
        )

    return agg


_edge_agg_h = _make_edge_agg_split(D_HALF, CHUNKS_SPLIT, tc_tiling=False)
_edge_agg_c = _make_edge_agg(C_PAD, CHUNKS, tc_tiling=False)

_BM = 1024


def _mm1(x, w):
    def body(x_ref, w_ref, o_ref):
        o_ref[0] = jnp.dot(x_ref[...], w_ref[0], preferred_element_type=jnp.float32)

    return pl.pallas_call(
        body,
        grid=(2, N_PAD // _BM),
        in_specs=[
            pl.BlockSpec((_BM, D_FEAT), lambda c, i: (i, 0)),
            pl.BlockSpec((1, D_FEAT, D_HALF), lambda c, i: (c, 0, 0)),
        ],
        out_specs=pl.BlockSpec((1, _BM, D_HALF), lambda c, i: (c, i, 0)),
        out_shape=jax.ShapeDtypeStruct((2, N_PAD, D_HALF), jnp.float32),
    )(x, w)


def _relu_mm2(p, b1, w2p):
    def body(p_ref, b_ref, w_ref, o_ref):
        agg1 = jnp.concatenate([p_ref[0], p_ref[1]], axis=-1)
        h = jnp.maximum(agg1 + b_ref[...], 0.0)
        o_ref[...] = jnp.dot(h, w_ref[...], preferred_element_type=jnp.float32)

    return pl.pallas_call(
        body,
        grid=(N_PAD // _BM,),
        in_specs=[
            pl.BlockSpec((2, _BM, D_HALF), lambda i: (0, i, 0)),
            pl.BlockSpec((1, HIDDEN), lambda i: (0, 0)),
            pl.BlockSpec((HIDDEN, C_PAD), lambda i: (0, 0)),
        ],
        out_specs=pl.BlockSpec((_BM, C_PAD), lambda i: (i, 0)),
        out_shape=jax.ShapeDtypeStruct((N_PAD, C_PAD), jnp.float32),
    )(p, b1.reshape(1, HIDDEN), w2p)


def _log_softmax(q, b2):
    bm = 1000

    def body(q_ref, b_ref, o_ref):
        s = q_ref[0] + q_ref[1]
        logits = s[:, :N_CLASSES] + b_ref[...]
        m = jnp.max(logits, axis=1, keepdims=True)
        lse = jnp.log(jnp.sum(jnp.exp(logits - m), axis=1, keepdims=True)) + m
        o_ref[...] = logits - lse

    return pl.pallas_call(
        body,
        grid=(N_NODES // bm,),
        in_specs=[
            pl.BlockSpec((2, bm, C_PAD), lambda i: (0, i, 0)),
            pl.BlockSpec((1, N_CLASSES), lambda i: (0, 0)),
        ],
        out_specs=pl.BlockSpec((bm, N_CLASSES), lambda i: (i, 0)),
        out_shape=jax.ShapeDtypeStruct((N_NODES, N_CLASSES), jnp.float32),
    )(q, b2.reshape(1, N_CLASSES))


def kernel(x, edge_index, edge_weight, W1, b1, W2, b2):
    src = edge_index[0].astype(jnp.int32)
    dst = edge_index[1].astype(jnp.int32)
    pad = E_PAD - src.shape[0]
    src_f = jnp.pad(src, (0, pad))
    dst_f = jnp.pad(dst, (0, pad))
    ew_f = jnp.pad(edge_weight.astype(jnp.float32), (0, pad))

    shp_w = (NW, CHUNKS, K_EDGE)
    shp_wg = (NW, CHUNKS // G_CH, G_CH, K_EDGE)
    shp_s = (N_SUB, CHUNKS_SPLIT, K_EDGE)
    shp_sg = (N_SUB, CHUNKS_SPLIT // G_CH, G_CH, K_EDGE)

    x_p = jnp.pad(x, ((0, N_PAD - N_NODES), (0, 0)))
    w1_s = W1.reshape(D_FEAT, 2, D_HALF).transpose(1, 0, 2)
    h1 = _mm1(x_p, w1_s)
    p1 = _edge_agg_h(
        h1, src_f.reshape(shp_s), dst_f.reshape(shp_sg), ew_f.reshape(shp_sg)
    )
    w2p = jnp.pad(W2, ((0, 0), (0, C_PAD - N_CLASSES)))
    h2 = _relu_mm2(p1, b1, w2p)
    p2 = _edge_agg_c(
        h2, src_f.reshape(shp_w), dst_f.reshape(shp_wg), ew_f.reshape(shp_wg)
    )
    return _log_softmax(p2, b2)


# speedup vs baseline: 1.6305x; 1.6305x over previous
import functools

import jax
import jax.numpy as jnp
from jax import lax
from jax.experimental import pallas as pl
from jax.experimental.pallas import tpu as pltpu
import jax.experimental.pallas.tpu_sc as plsc

N_NODES = 10000
N_PAD = 10240
D_FEAT = 128
HIDDEN = 128
D_HALF = HIDDEN // 2
N_CLASSES = 40
C_PAD = 48

N_SUB = 16
NW = 2 * N_SUB
K_EDGE = 128
G_CH = 8
CHUNKS = 80
E_PAD = NW * K_EDGE * CHUNKS
CHUNKS_SPLIT = E_PAD // (N_SUB * K_EDGE)

ROWS_PER_SUB = N_PAD // N_SUB


def _zero_acc_stripe(rows, acc, sid, d):
    zero16 = jnp.zeros((16,), jnp.float32)

    def zrow(i, carry):
        for j in range(d // 16):
            rows[0, i, pl.ds(j * 16, 16)] = zero16
        return carry

    lax.fori_loop(0, K_EDGE, zrow, 0)
    for t in range(ROWS_PER_SUB // K_EDGE):
        pltpu.sync_copy(
            rows.at[0],
            acc.at[pl.ds(sid * ROWS_PER_SUB + t * K_EDGE, K_EDGE)],
        )


def _edge_loop(chunks, d, eid, h_sp, acc, dst_hbm, ew_hbm,
               isrc, dbuf, wbuf, rows, gsems, msems, wsems, ssems):
    pltpu.sync_copy(dst_hbm.at[eid, 0], dbuf.at[0])
    pltpu.sync_copy(ew_hbm.at[eid, 0], wbuf.at[0])
    pltpu.async_copy(h_sp.at[isrc.at[0]], rows.at[0], gsems.at[0])

    @pl.loop(0, chunks, step=2)
    def chunk2(c0):
        for b in range(2):
            c = c0 + b
            nxt = c + 1

            @pl.when(nxt < chunks)
            def _():
                pltpu.async_copy(
                    h_sp.at[isrc.at[nxt]], rows.at[1 - b], gsems.at[1 - b]
                )
                pltpu.async_copy(
                    dst_hbm.at[eid, nxt], dbuf.at[1 - b], msems.at[1 - b]
                )
                pltpu.async_copy(
                    ew_hbm.at[eid, nxt], wbuf.at[1 - b], wsems.at[1 - b]
                )

            pltpu.make_async_copy(
                h_sp.at[isrc.at[c]], rows.at[b], gsems.at[b]
            ).wait()

            @pl.when(c > 0)
            def _():
                pltpu.make_async_copy(
                    dst_hbm.at[eid, c], dbuf.at[b], msems.at[b]
                ).wait()
                pltpu.make_async_copy(
                    ew_hbm.at[eid, c], wbuf.at[b], wsems.at[b]
                ).wait()

            for grp in range(K_EDGE // 16):
                wv = wbuf[b, pl.ds(grp * 16, 16)]
                for ii in range(16):
                    wb = wv.at[jnp.full((16,), ii, jnp.int32)].get(
                        mode="promise_in_bounds"
                    )
                    for j in range(d // 16):
                        rows[b, grp * 16 + ii, pl.ds(j * 16, 16)] = (
                            rows[b, grp * 16 + ii, pl.ds(j * 16, 16)] * wb
                        )

            pltpu.sync_copy(rows.at[b], acc.at[dbuf.at[b]], add=True)


def _edge_scratch(d, chunks):
    return [
        pltpu.VMEM((chunks, K_EDGE), jnp.int32),
        pltpu.VMEM((2, K_EDGE), jnp.int32),
        pltpu.VMEM((2, K_EDGE), jnp.float32),
        pltpu.VMEM((2, K_EDGE, d), jnp.float32),
        pltpu.SemaphoreType.DMA((2,)),
        pltpu.SemaphoreType.DMA((2,)),
        pltpu.SemaphoreType.DMA((2,)),
        pltpu.SemaphoreType.DMA((2,)),
    ]


def _make_edge_agg_split(dch, chunks, tc_tiling):
    mesh = plsc.VectorSubcoreMesh(core_axis_name="c", subcore_axis_name="s")

    @functools.partial(
        pl.kernel,
        out_type=jax.ShapeDtypeStruct((2, N_PAD, dch), jnp.float32),
        mesh=mesh,
        scratch_types=[
            pltpu.VMEM_SHARED((N_PAD, dch), jnp.float32),
            pltpu.VMEM_SHARED((N_PAD, dch), jnp.float32),
        ] + _edge_scratch(dch, chunks),
        compiler_params=pltpu.CompilerParams(
            needs_layout_passes=False, use_tc_tiling_on_sc=tc_tiling
        ),
    )
    def agg(h_hbm, src_hbm, dst_hbm, ew_hbm, out_hbm,
            acc, h_sp, isrc, dbuf, wbuf, rows, gsems, msems, wsems, ssems):
        cid = lax.axis_index("c")
        sid = lax.axis_index("s")

        pltpu.sync_copy(src_hbm.at[sid], isrc)
        pltpu.sync_copy(
            h_hbm.at[cid, pl.ds(sid * ROWS_PER_SUB, ROWS_PER_SUB)],
            h_sp.at[pl.ds(sid * ROWS_PER_SUB, ROWS_PER_SUB)],
        )
        _zero_acc_stripe(rows, acc, sid, dch)
        plsc.subcore_barrier()

        _edge_loop(chunks, dch, sid, h_sp, acc, dst_hbm, ew_hbm,
                   isrc, dbuf, wbuf, rows, gsems, msems, wsems, ssems)

        plsc.subcore_barrier()
        pltpu.sync_copy(
            acc.at[pl.ds(sid * ROWS_PER_SUB, ROWS_PER_SUB)],
            out_hbm.at[cid, pl.ds(sid * ROWS_PER_SUB, ROWS_PER_SUB)],
        )

    return agg


def _make_edge_agg(d, chunks, tc_tiling):
    mesh = plsc.VectorSubcoreMesh(core_axis_name="c", subcore_axis_name="s")

    @functools.partial(
        pl.kernel,
        out_type=jax.ShapeDtypeStruct((2, N_PAD, d), jnp.float32),
        mesh=mesh,
        scratch_types=[
            pltpu.VMEM_SHARED((N_PAD, d), jnp.float32),
            pltpu.VMEM_SHARED((N_PAD, d), jnp.float32),
        ] + _edge_scratch(d, chunks),
        compiler_params=pltpu.CompilerParams(
            needs_layout_passes=False, use_tc_tiling_on_sc=tc_tiling
        ),
    )
    def agg(h_hbm, src_hbm, dst_hbm, ew_hbm, out_hbm,
            acc, h_sp, isrc, dbuf, wbuf, rows, gsems, msems, wsems, ssems):
        cid = lax.axis_index("c")
        sid = lax.axis_index("s")
        wid = cid * N_SUB + sid

        pltpu.sync_copy(src_hbm.at[wid], isrc)
        pltpu.sync_copy(
            h_hbm.at[pl.ds(sid * ROWS_PER_SUB, ROWS_PER_SUB)],
            h_sp.at[pl.ds(sid * ROWS_PER_SUB, ROWS_PER_SUB)],
        )
        _zero_acc_stripe(rows, acc, sid, d)
        plsc.subcore_barrier()

        _edge_loop(chunks, d, wid, h_sp, acc, dst_hbm, ew_hbm,
                   isrc, dbuf, wbuf, rows, gsems, msems, wsems, ssems)

        plsc.subcore_barrier()
        pltpu.sync_copy(
            acc.at[pl.ds(sid * ROWS_PER_SUB, ROWS_PER_SUB)],
            out_hbm.at[cid, pl.ds(sid * ROWS_PER_SUB, ROWS_PER_SUB)],
        )

    return agg


_edge_agg_h = _make_edge_agg_split(D_HALF, CHUNKS_SPLIT, tc_tiling=False)
_edge_agg_c = _make_edge_agg(C_PAD, CHUNKS, tc_tiling=False)

_BM = 1024


def _mm1(x, w):
    def body(x_ref, w_ref, o_ref):
        o_ref[0] = jnp.dot(x_ref[...], w_ref[0], preferred_element_type=jnp.float32)

    return pl.pallas_call(
        body,
        grid=(2, N_PAD // _BM),
        in_specs=[
            pl.BlockSpec((_BM, D_FEAT), lambda c, i: (i, 0)),
            pl.BlockSpec((1, D_FEAT, D_HALF), lambda c, i: (c, 0, 0)),
        ],
        out_specs=pl.BlockSpec((1, _BM, D_HALF), lambda c, i: (c, i, 0)),
        out_shape=jax.ShapeDtypeStruct((2, N_PAD, D_HALF), jnp.float32),
    )(x, w)


def _relu_mm2(p, b1, w2p):
    def body(p_ref, b_ref, w_ref, o_ref):
        agg1 = jnp.concatenate([p_ref[0], p_ref[1]], axis=-1)
        h = jnp.maximum(agg1 + b_ref[...], 0.0)
        o_ref[...] = jnp.dot(h, w_ref[...], preferred_element_type=jnp.float32)

    return pl.pallas_call(
        body,
        grid=(N_PAD // _BM,),
        in_specs=[
            pl.BlockSpec((2, _BM, D_HALF), lambda i: (0, i, 0)),
            pl.BlockSpec((1, HIDDEN), lambda i: (0, 0)),
            pl.BlockSpec((HIDDEN, C_PAD), lambda i: (0, 0)),
        ],
        out_specs=pl.BlockSpec((_BM, C_PAD), lambda i: (i, 0)),
        out_shape=jax.ShapeDtypeStruct((N_PAD, C_PAD), jnp.float32),
    )(p, b1.reshape(1, HIDDEN), w2p)


def _log_softmax(q, b2):
    bm = 1000

    def body(q_ref, b_ref, o_ref):
        s = q_ref[0] + q_ref[1]
        logits = s[:, :N_CLASSES] + b_ref[...]
        m = jnp.max(logits, axis=1, keepdims=True)
        lse = jnp.log(jnp.sum(jnp.exp(logits - m), axis=1, keepdims=True)) + m
        o_ref[...] = logits - lse

    return pl.pallas_call(
        body,
        grid=(N_NODES // bm,),
        in_specs=[
            pl.BlockSpec((2, bm, C_PAD), lambda i: (0, i, 0)),
            pl.BlockSpec((1, N_CLASSES), lambda i: (0, 0)),
        ],
        out_specs=pl.BlockSpec((bm, N_CLASSES), lambda i: (i, 0)),
        out_shape=jax.ShapeDtypeStruct((N_NODES, N_CLASSES), jnp.float32),
    )(q, b2.reshape(1, N_CLASSES))


def kernel(x, edge_index, edge_weight, W1, b1, W2, b2):
    src = edge_index[0].astype(jnp.int32)
    dst = edge_index[1].astype(jnp.int32)
    pad = E_PAD - src.shape[0]
    src_f = jnp.pad(src, (0, pad))
    dst_f = jnp.pad(dst, (0, pad))
    ew_f = jnp.pad(edge_weight.astype(jnp.float32), (0, pad))

    shp_w = (NW, CHUNKS, K_EDGE)
    shp_s = (N_SUB, CHUNKS_SPLIT, K_EDGE)

    x_p = jnp.pad(x, ((0, N_PAD - N_NODES), (0, 0)))
    w1_s = W1.reshape(D_FEAT, 2, D_HALF).transpose(1, 0, 2)
    h1 = _mm1(x_p, w1_s)
    p1 = _edge_agg_h(
        h1, src_f.reshape(shp_s), dst_f.reshape(shp_s), ew_f.reshape(shp_s)
    )
    w2p = jnp.pad(W2, ((0, 0), (0, C_PAD - N_CLASSES)))
    h2 = _relu_mm2(p1, b1, w2p)
    p2 = _edge_agg_c(
        h2, src_f.reshape(shp_w), dst_f.reshape(shp_w), ew_f.reshape(shp_w)
    )
    return _log_softmax(p2, b2)
